# baseline (speedup 1.0000x reference)
import functools

import numpy as np
import jax
import jax.numpy as jnp
from jax import lax
from jax.experimental import pallas as pl
from jax.experimental.pallas import tpu as pltpu
from jax.experimental.pallas import tpu_sc as plsc

B = 4096
NT = 26
VOCAB = 100000
DIM = 32
NI = NT + 1
TOTAL = B * NT
NW = 32
PER_W = TOTAL // NW
CHUNK = 128
NCH = PER_W // CHUNK

_LI, _LJ = np.tril_indices(NI, k=-1)
_POS = np.asarray(_LI * NI + _LJ, dtype=np.int32)

BB = 512
GRID = B // BB


def _sc_gather(table128, gid, off):
    mesh = plsc.VectorSubcoreMesh(core_axis_name="c", subcore_axis_name="s")

    @functools.partial(
        pl.kernel,
        mesh=mesh,
        compiler_params=pltpu.CompilerParams(needs_layout_passes=False),
        out_type=jax.ShapeDtypeStruct((TOTAL * DIM,), jnp.float32),
        scratch_types=[
            pltpu.VMEM((PER_W,), jnp.int32),
            pltpu.VMEM((PER_W,), jnp.int32),
            pltpu.VMEM((2 * CHUNK, 128), jnp.float32),
            pltpu.VMEM((CHUNK * DIM,), jnp.float32),
            pltpu.SemaphoreType.DMA,
            pltpu.SemaphoreType.DMA,
        ],
    )
    def run(t_hbm, gid_hbm, off_hbm, out_hbm,
            gid_v, off_v, g_ring, out_c, sem0, sem1):
        wid = lax.axis_index("s") * 2 + lax.axis_index("c")
        base = wid * PER_W
        pltpu.sync_copy(gid_hbm.at[pl.ds(base, PER_W)], gid_v)
        pltpu.sync_copy(off_hbm.at[pl.ds(base, PER_W)], off_v)

        def fire(j, slot):
            src = t_hbm.at[gid_v.at[pl.ds(j * CHUNK, CHUNK)]]

            @pl.when(slot == 0)
            def _():
                pltpu.async_copy(src, g_ring.at[pl.ds(0, CHUNK)], sem0)

            @pl.when(slot == 1)
            def _():
                pltpu.async_copy(src, g_ring.at[pl.ds(CHUNK, CHUNK)], sem1)

        fire(0, 0)

        def body(j, c):
            slot = lax.rem(j, 2)

            @pl.when(j + 1 < NCH)
            def _():
                fire(j + 1, lax.rem(j + 1, 2))

            @pl.when(slot == 0)
            def _():
                pltpu.make_async_copy(
                    t_hbm.at[pl.ds(0, CHUNK)], g_ring.at[pl.ds(0, CHUNK)],
                    sem0).wait()

            @pl.when(slot == 1)
            def _():
                pltpu.make_async_copy(
                    t_hbm.at[pl.ds(0, CHUNK)], g_ring.at[pl.ds(CHUNK, CHUNK)],
                    sem1).wait()

            rowbase = slot * CHUNK

            def rg_body(rg, c2):
                lane = lax.iota(jnp.int32, 16)
                off16 = off_v[pl.ds(j * CHUNK + rg * 16, 16)]
                rowmaj = rowbase + rg * 16 + lane
                posbase = (rg * 16 + lane) * DIM
                for d in range(DIM):
                    val = plsc.load_gather(g_ring, [rowmaj, off16 + d])
                    plsc.store_scatter(out_c, [posbase + d], val)
                return c2

            lax.fori_loop(0, CHUNK // 16, rg_body, 0)
            pltpu.sync_copy(
                out_c,
                out_hbm.at[pl.ds((base + j * CHUNK) * DIM, CHUNK * DIM)])
            return c

        lax.fori_loop(0, NCH, body, 0)

    return run(table128, gid, off)


def _tc_body(num_ref, emb_ref, wb0, bb0, wb1, bb1, wb2, bb2,
             w0x, w0f, bt0, wt1, bt1, wt2, bt2, wt3, bt3, wt4, bt4, out_ref):
    dot = lambda a, b: lax.dot_general(
        a, b, (((1,), (0,)), ((), ())), preferred_element_type=jnp.float32)
    x = num_ref[...]
    x = jnp.maximum(dot(x, wb0[...]) + bb0[...], 0.0)
    x = jnp.maximum(dot(x, wb1[...]) + bb1[...], 0.0)
    x = jnp.maximum(dot(x, wb2[...]) + bb2[...], 0.0)
    feats = jnp.concatenate([x, emb_ref[...]], axis=1)
    f3 = feats.reshape(BB, NI, DIM)
    xact = lax.dot_general(
        f3, f3, (((2,), (2,)), ((0,), (0,))),
        preferred_element_type=jnp.float32)
    xflat = xact.reshape(BB, NI * NI)
    z = jnp.maximum(dot(x, w0x[...]) + dot(xflat, w0f[...]) + bt0[...], 0.0)
    z = jnp.maximum(dot(z, wt1[...]) + bt1[...], 0.0)
    z = jnp.maximum(dot(z, wt2[...]) + bt2[...], 0.0)
    z = jnp.maximum(dot(z, wt3[...]) + bt3[...], 0.0)
    out_ref[...] = dot(z, wt4[...]) + bt4[...]


def _tc_forward(num, emb2, wb0, bb0, wb1, bb1, wb2, bb2,
                w0x, w0f, bt0, wt1, bt1, wt2, bt2, wt3, bt3, wt4, bt4):
    full = lambda a: pl.BlockSpec(a.shape, lambda i: (0,) * a.ndim)
    weights = (wb0, bb0, wb1, bb1, wb2, bb2, w0x, w0f, bt0,
               wt1, bt1, wt2, bt2, wt3, bt3, wt4, bt4)
    return pl.pallas_call(
        _tc_body,
        grid=(GRID,),
        in_specs=[
            pl.BlockSpec((BB, num.shape[1]), lambda i: (i, 0)),
            pl.BlockSpec((BB, emb2.shape[1]), lambda i: (i, 0)),
            *[full(w) for w in weights],
        ],
        out_specs=pl.BlockSpec((BB, 1), lambda i: (i, 0)),
        out_shape=jax.ShapeDtypeStruct((B, 1), jnp.float32),
    )(num, emb2, *weights)


def kernel(numerical_features, categorical_features, embedding_tables,
           Wb0, bb0, Wb1, bb1, Wb2, bb2,
           Wt0, bt0, Wt1, bt1, Wt2, bt2, Wt3, bt3, Wt4, bt4):
    offs = (jnp.arange(NT, dtype=jnp.int32) * VOCAB)[None, :]
    idx_flat = (categorical_features + offs).reshape(TOTAL)
    gid = idx_flat >> 2
    off = (idx_flat & 3) << 5
    table128 = embedding_tables.reshape(NT * VOCAB // 4, 4 * DIM)
    emb = _sc_gather(table128, gid, off)
    emb2 = emb.reshape(B, NT * DIM)

    w0x = Wt0[:DIM]
    w0f = jnp.zeros((NI * NI, Wt0.shape[1]), jnp.float32).at[_POS].set(Wt0[DIM:])
    r1 = lambda v: v.reshape(1, -1)
    return _tc_forward(
        numerical_features, emb2, Wb0, r1(bb0), Wb1, r1(bb1), Wb2, r1(bb2),
        w0x, w0f, r1(bt0), Wt1, r1(bt1), Wt2, r1(bt2), Wt3, r1(bt3),
        Wt4, r1(bt4))


# speedup vs baseline: 1.8158x; 1.8158x over previous
import functools

import numpy as np
import jax
import jax.numpy as jnp
from jax import lax
from jax.experimental import pallas as pl
from jax.experimental.pallas import tpu as pltpu
from jax.experimental.pallas import tpu_sc as plsc

B = 4096
NT = 26
VOCAB = 100000
DIM = 32
NI = NT + 1
TOTAL = B * NT
NW = 32
PER_W = TOTAL // NW
CHUNK = 128
NCH = PER_W // CHUNK

_LI, _LJ = np.tril_indices(NI, k=-1)
_POS = np.asarray(_LI * NI + _LJ, dtype=np.int32)

BB = 512
GRID = B // BB


NROWS = NT * DIM
ROWS_W = NROWS // NW
SEG = 128
NSEG = B // SEG


def _sc_gather(tflat, idx_all):
    mesh = plsc.VectorSubcoreMesh(core_axis_name="c", subcore_axis_name="s")

    @functools.partial(
        pl.kernel,
        mesh=mesh,
        out_type=jax.ShapeDtypeStruct((NROWS * B,), jnp.float32),
        scratch_types=[
            pltpu.VMEM((B,), jnp.int32),
            pltpu.VMEM((B,), jnp.int32),
            pltpu.VMEM((ROWS_W * B,), jnp.float32),
            pltpu.SemaphoreType.DMA,
        ],
    )
    def run(t_hbm, idx_hbm, out_hbm, idx_a, idx_b, dst, sem):
        wid = lax.axis_index("s") * 2 + lax.axis_index("c")
        rbase = wid * ROWS_W

        def fire_from(buf, i):
            for seg in range(NSEG):
                pltpu.async_copy(
                    t_hbm.at[buf.at[pl.ds(seg * SEG, SEG)]],
                    dst.at[pl.ds(i * B + seg * SEG, SEG)],
                    sem,
                )

        def drain_row():
            pltpu.make_async_copy(
                t_hbm.at[pl.ds(0, B)], dst.at[pl.ds(0, B)], sem).wait()

        def row_body(i, c):
            par = lax.rem(i, 2)

            @pl.when(par == 0)
            def _():
                pltpu.sync_copy(idx_hbm.at[pl.ds((rbase + i) * B, B)], idx_a)

            @pl.when(par == 1)
            def _():
                pltpu.sync_copy(idx_hbm.at[pl.ds((rbase + i) * B, B)], idx_b)

            @pl.when(i >= 1)
            def _():
                drain_row()

            @pl.when(par == 0)
            def _():
                fire_from(idx_a, i)

            @pl.when(par == 1)
            def _():
                fire_from(idx_b, i)

            return c

        lax.fori_loop(0, ROWS_W, row_body, 0)
        drain_row()
        pltpu.sync_copy(dst, out_hbm.at[pl.ds(rbase * B, ROWS_W * B)])

    return run(tflat, idx_all)


def _tc_body(num_ref, emb_ref, wb0, bb0, wb1, bb1, wb2, bb2,
             w0x, w0f, bt0, wt1, bt1, wt2, bt2, wt3, bt3, wt4, bt4, out_ref):
    dot = lambda a, b: lax.dot_general(
        a, b, (((1,), (0,)), ((), ())), preferred_element_type=jnp.float32)
    x = num_ref[...]
    x = jnp.maximum(dot(x, wb0[...]) + bb0[...], 0.0)
    x = jnp.maximum(dot(x, wb1[...]) + bb1[...], 0.0)
    x = jnp.maximum(dot(x, wb2[...]) + bb2[...], 0.0)
    feats = jnp.concatenate([x, emb_ref[...]], axis=1)
    f3 = feats.reshape(BB, NI, DIM)
    xact = lax.dot_general(
        f3, f3, (((2,), (2,)), ((0,), (0,))),
        preferred_element_type=jnp.float32)
    xflat = xact.reshape(BB, NI * NI)
    z = jnp.maximum(dot(x, w0x[...]) + dot(xflat, w0f[...]) + bt0[...], 0.0)
    z = jnp.maximum(dot(z, wt1[...]) + bt1[...], 0.0)
    z = jnp.maximum(dot(z, wt2[...]) + bt2[...], 0.0)
    z = jnp.maximum(dot(z, wt3[...]) + bt3[...], 0.0)
    out_ref[...] = dot(z, wt4[...]) + bt4[...]


def _tc_forward(num, emb2, wb0, bb0, wb1, bb1, wb2, bb2,
                w0x, w0f, bt0, wt1, bt1, wt2, bt2, wt3, bt3, wt4, bt4):
    full = lambda a: pl.BlockSpec(a.shape, lambda i: (0,) * a.ndim)
    weights = (wb0, bb0, wb1, bb1, wb2, bb2, w0x, w0f, bt0,
               wt1, bt1, wt2, bt2, wt3, bt3, wt4, bt4)
    return pl.pallas_call(
        _tc_body,
        grid=(GRID,),
        in_specs=[
            pl.BlockSpec((BB, num.shape[1]), lambda i: (i, 0)),
            pl.BlockSpec((BB, emb2.shape[1]), lambda i: (i, 0)),
            *[full(w) for w in weights],
        ],
        out_specs=pl.BlockSpec((BB, 1), lambda i: (i, 0)),
        out_shape=jax.ShapeDtypeStruct((B, 1), jnp.float32),
    )(num, emb2, *weights)


def kernel(numerical_features, categorical_features, embedding_tables,
           Wb0, bb0, Wb1, bb1, Wb2, bb2,
           Wt0, bt0, Wt1, bt1, Wt2, bt2, Wt3, bt3, Wt4, bt4):
    rowbase = (jnp.arange(NROWS, dtype=jnp.int32) * VOCAB)[:, None]
    idx_all = (rowbase
               + jnp.repeat(categorical_features.T, DIM, axis=0)).reshape(-1)
    tflat = jnp.swapaxes(embedding_tables, 1, 2).reshape(-1)
    emb_dm = _sc_gather(tflat, idx_all)
    emb2 = emb_dm.reshape(NROWS, B).T

    w0x = Wt0[:DIM]
    w0f = jnp.zeros((NI * NI, Wt0.shape[1]), jnp.float32).at[_POS].set(Wt0[DIM:])
    r1 = lambda v: v.reshape(1, -1)
    return _tc_forward(
        numerical_features, emb2, Wb0, r1(bb0), Wb1, r1(bb1), Wb2, r1(bb2),
        w0x, w0f, r1(bt0), Wt1, r1(bt1), Wt2, r1(bt2), Wt3, r1(bt3),
        Wt4, r1(bt4))


# speedup vs baseline: 1.8191x; 1.0018x over previous
import functools

import numpy as np
import jax
import jax.numpy as jnp
from jax import lax
from jax.experimental import pallas as pl
from jax.experimental.pallas import tpu as pltpu
from jax.experimental.pallas import tpu_sc as plsc

B = 4096
NT = 26
VOCAB = 100000
DIM = 32
NI = NT + 1
TOTAL = B * NT
NW = 32
PER_W = TOTAL // NW
CHUNK = 128
NCH = PER_W // CHUNK

_LI, _LJ = np.tril_indices(NI, k=-1)
_POS = np.asarray(_LI * NI + _LJ, dtype=np.int32)

BB = 512
GRID = B // BB


NROWS = NT * DIM
ROWS_W = NROWS // NW
SEG = 128
NSEG = B // SEG


def _sc_gather(tflat, idx_all):
    mesh = plsc.VectorSubcoreMesh(core_axis_name="c", subcore_axis_name="s")

    HALF = B // 2
    NU = ROWS_W * 2

    @functools.partial(
        pl.kernel,
        mesh=mesh,
        out_type=jax.ShapeDtypeStruct((NROWS * B,), jnp.float32),
        scratch_types=[
            pltpu.VMEM((HALF,), jnp.int32),
            pltpu.VMEM((HALF,), jnp.int32),
            pltpu.VMEM((ROWS_W * B,), jnp.float32),
            pltpu.SemaphoreType.DMA,
            pltpu.SemaphoreType.DMA,
        ],
    )
    def run(t_hbm, idx_hbm, out_hbm, idx_a, idx_b, dst, sem0, sem1):
        wid = lax.axis_index("s") * 2 + lax.axis_index("c")
        ibase = wid * ROWS_W * B

        def drain(sem):
            pltpu.make_async_copy(
                t_hbm.at[pl.ds(0, HALF)], dst.at[pl.ds(0, HALF)], sem).wait()

        def step(u, buf, sem):
            pltpu.sync_copy(idx_hbm.at[pl.ds(ibase + u * HALF, HALF)], buf)
            for seg in range(HALF // SEG):
                pltpu.async_copy(
                    t_hbm.at[buf.at[pl.ds(seg * SEG, SEG)]],
                    dst.at[pl.ds(u * HALF + seg * SEG, SEG)],
                    sem,
                )

        def unit_body(u, c):
            par = lax.rem(u, 2)

            @pl.when(par == 0)
            def _():
                @pl.when(u >= 2)
                def _():
                    drain(sem0)
                step(u, idx_a, sem0)

            @pl.when(par == 1)
            def _():
                @pl.when(u >= 2)
                def _():
                    drain(sem1)
                step(u, idx_b, sem1)

            return c

        lax.fori_loop(0, NU, unit_body, 0)
        drain(sem0)
        drain(sem1)
        pltpu.sync_copy(dst, out_hbm.at[pl.ds(ibase, ROWS_W * B)])

    return run(tflat, idx_all)


def _tc_body(num_ref, emb_ref, wb0, bb0, wb1, bb1, wb2, bb2,
             w0x, w0f, bt0, wt1, bt1, wt2, bt2, wt3, bt3, wt4, bt4, out_ref):
    bf = jnp.bfloat16
    dot = lambda a, b: lax.dot_general(
        a.astype(bf), b, (((1,), (0,)), ((), ())),
        preferred_element_type=jnp.float32)
    x = num_ref[...]
    x = jnp.maximum(dot(x, wb0[...]) + bb0[...], 0.0)
    x = jnp.maximum(dot(x, wb1[...]) + bb1[...], 0.0)
    x = jnp.maximum(dot(x, wb2[...]) + bb2[...], 0.0)
    feats = jnp.concatenate([x.astype(bf), emb_ref[...]], axis=1)
    f3 = feats.reshape(BB, NI, DIM)
    xact = lax.dot_general(
        f3, f3, (((2,), (2,)), ((0,), (0,))),
        preferred_element_type=jnp.float32)
    xflat = xact.reshape(BB, NI * NI)
    z = jnp.maximum(dot(x, w0x[...]) + dot(xflat, w0f[...]) + bt0[...], 0.0)
    z = jnp.maximum(dot(z, wt1[...]) + bt1[...], 0.0)
    z = jnp.maximum(dot(z, wt2[...]) + bt2[...], 0.0)
    z = jnp.maximum(dot(z, wt3[...]) + bt3[...], 0.0)
    out_ref[...] = dot(z, wt4[...]) + bt4[...]


def _tc_forward(num, emb2, wb0, bb0, wb1, bb1, wb2, bb2,
                w0x, w0f, bt0, wt1, bt1, wt2, bt2, wt3, bt3, wt4, bt4):
    full = lambda a: pl.BlockSpec(a.shape, lambda i: (0,) * a.ndim)
    weights = (wb0, bb0, wb1, bb1, wb2, bb2, w0x, w0f, bt0,
               wt1, bt1, wt2, bt2, wt3, bt3, wt4, bt4)
    return pl.pallas_call(
        _tc_body,
        grid=(GRID,),
        in_specs=[
            pl.BlockSpec((BB, num.shape[1]), lambda i: (i, 0)),
            pl.BlockSpec((BB, emb2.shape[1]), lambda i: (i, 0)),
            *[full(w) for w in weights],
        ],
        out_specs=pl.BlockSpec((BB, 1), lambda i: (i, 0)),
        out_shape=jax.ShapeDtypeStruct((B, 1), jnp.float32),
    )(num, emb2, *weights)


def kernel(numerical_features, categorical_features, embedding_tables,
           Wb0, bb0, Wb1, bb1, Wb2, bb2,
           Wt0, bt0, Wt1, bt1, Wt2, bt2, Wt3, bt3, Wt4, bt4):
    rowbase = (jnp.arange(NROWS, dtype=jnp.int32) * VOCAB)[:, None]
    idx_all = (rowbase
               + jnp.repeat(categorical_features.T, DIM, axis=0)).reshape(-1)
    tflat = jnp.swapaxes(embedding_tables, 1, 2).reshape(-1)
    emb_dm = _sc_gather(tflat, idx_all)
    bf = jnp.bfloat16
    emb2 = emb_dm.reshape(NROWS, B).T.astype(bf)

    w0x = Wt0[:DIM]
    w0f = jnp.zeros((NI * NI, Wt0.shape[1]), jnp.float32).at[_POS].set(Wt0[DIM:])
    r1 = lambda v: v.reshape(1, -1)
    return _tc_forward(
        numerical_features, emb2,
        Wb0.astype(bf), r1(bb0), Wb1.astype(bf), r1(bb1), Wb2.astype(bf),
        r1(bb2), w0x.astype(bf), w0f.astype(bf), r1(bt0), Wt1.astype(bf),
        r1(bt1), Wt2.astype(bf), r1(bt2), Wt3.astype(bf), r1(bt3),
        Wt4.astype(bf), r1(bt4))


# speedup vs baseline: 2.6541x; 1.4590x over previous
import functools

import numpy as np
import jax
import jax.numpy as jnp
from jax import lax
from jax.experimental import pallas as pl
from jax.experimental.pallas import tpu as pltpu
from jax.experimental.pallas import tpu_sc as plsc

B = 4096
NT = 26
VOCAB = 100000
DIM = 32
NI = NT + 1
TOTAL = B * NT
NW = 32
PER_W = TOTAL // NW
CHUNK = 128
NCH = PER_W // CHUNK

_LI, _LJ = np.tril_indices(NI, k=-1)
_POS = np.asarray(_LI * NI + _LJ, dtype=np.int32)

BB = 512
GRID = B // BB


NROWS = NT * DIM
ROWS_W = NROWS // NW
SEG = 128
NSEG = B // SEG
VROW = VOCAB // 128 * 128
VOCABP = (VOCAB + 127) // 128 * 128
PROWS = VOCABP // 128


def _tc_repack(tabT):
    def body(in_ref, out_ref):
        x = in_ref[0]
        for j in range(8):
            row = x[j]
            out_ref[j * PROWS:j * PROWS + VROW // 128, :] = (
                row[:VROW].reshape(VROW // 128, 128))
            out_ref[j * PROWS + VROW // 128, :VOCAB - VROW] = row[VROW:]

    return pl.pallas_call(
        body,
        grid=(NT, 4),
        in_specs=[pl.BlockSpec((1, 8, VOCAB), lambda t, g: (t, g, 0))],
        out_specs=pl.BlockSpec((8 * PROWS, 128), lambda t, g: (t * 4 + g, 0)),
        out_shape=jax.ShapeDtypeStruct((NROWS * PROWS, 128), jnp.float32),
    )(tabT)


def _sc_gather(tflat, idx_all):
    mesh = plsc.VectorSubcoreMesh(core_axis_name="c", subcore_axis_name="s")

    HALF = B // 2
    NU = ROWS_W * 2

    @functools.partial(
        pl.kernel,
        mesh=mesh,
        out_type=jax.ShapeDtypeStruct((NROWS * B,), jnp.float32),
        scratch_types=[
            pltpu.VMEM((HALF,), jnp.int32),
            pltpu.VMEM((HALF,), jnp.int32),
            pltpu.VMEM((ROWS_W * B,), jnp.float32),
            pltpu.SemaphoreType.DMA,
            pltpu.SemaphoreType.DMA,
        ],
    )
    def run(t_hbm, idx_hbm, out_hbm, idx_a, idx_b, dst, sem0, sem1):
        wid = lax.axis_index("s") * 2 + lax.axis_index("c")
        ibase = wid * ROWS_W * B

        def drain(sem):
            pltpu.make_async_copy(
                t_hbm.at[pl.ds(0, HALF)], dst.at[pl.ds(0, HALF)], sem).wait()

        def step(u, buf, sem):
            pltpu.sync_copy(idx_hbm.at[pl.ds(ibase + u * HALF, HALF)], buf)
            for seg in range(HALF // SEG):
                pltpu.async_copy(
                    t_hbm.at[buf.at[pl.ds(seg * SEG, SEG)]],
                    dst.at[pl.ds(u * HALF + seg * SEG, SEG)],
                    sem,
                )

        def unit_body(u, c):
            par = lax.rem(u, 2)

            @pl.when(par == 0)
            def _():
                @pl.when(u >= 2)
                def _():
                    drain(sem0)
                step(u, idx_a, sem0)

            @pl.when(par == 1)
            def _():
                @pl.when(u >= 2)
                def _():
                    drain(sem1)
                step(u, idx_b, sem1)

            return c

        lax.fori_loop(0, NU, unit_body, 0)
        drain(sem0)
        drain(sem1)
        pltpu.sync_copy(dst, out_hbm.at[pl.ds(ibase, ROWS_W * B)])

    return run(tflat, idx_all)


def _tc_body(num_ref, emb_ref, wb0, bb0, wb1, bb1, wb2, bb2,
             w0x, w0f, bt0, wt1, bt1, wt2, bt2, wt3, bt3, wt4, bt4, out_ref):
    bf = jnp.bfloat16
    dot = lambda a, b: lax.dot_general(
        a.astype(bf), b, (((1,), (0,)), ((), ())),
        preferred_element_type=jnp.float32)
    x = num_ref[...]
    x = jnp.maximum(dot(x, wb0[...]) + bb0[...], 0.0)
    x = jnp.maximum(dot(x, wb1[...]) + bb1[...], 0.0)
    x = jnp.maximum(dot(x, wb2[...]) + bb2[...], 0.0)
    feats = jnp.concatenate([x.astype(bf), emb_ref[...]], axis=1)
    f3 = feats.reshape(BB, NI, DIM)
    xact = lax.dot_general(
        f3, f3, (((2,), (2,)), ((0,), (0,))),
        preferred_element_type=jnp.float32)
    xflat = xact.reshape(BB, NI * NI)
    z = jnp.maximum(dot(x, w0x[...]) + dot(xflat, w0f[...]) + bt0[...], 0.0)
    z = jnp.maximum(dot(z, wt1[...]) + bt1[...], 0.0)
    z = jnp.maximum(dot(z, wt2[...]) + bt2[...], 0.0)
    z = jnp.maximum(dot(z, wt3[...]) + bt3[...], 0.0)
    out_ref[...] = dot(z, wt4[...]) + bt4[...]


def _tc_forward(num, emb2, wb0, bb0, wb1, bb1, wb2, bb2,
                w0x, w0f, bt0, wt1, bt1, wt2, bt2, wt3, bt3, wt4, bt4):
    full = lambda a: pl.BlockSpec(a.shape, lambda i: (0,) * a.ndim)
    weights = (wb0, bb0, wb1, bb1, wb2, bb2, w0x, w0f, bt0,
               wt1, bt1, wt2, bt2, wt3, bt3, wt4, bt4)
    return pl.pallas_call(
        _tc_body,
        grid=(GRID,),
        in_specs=[
            pl.BlockSpec((BB, num.shape[1]), lambda i: (i, 0)),
            pl.BlockSpec((BB, emb2.shape[1]), lambda i: (i, 0)),
            *[full(w) for w in weights],
        ],
        out_specs=pl.BlockSpec((BB, 1), lambda i: (i, 0)),
        out_shape=jax.ShapeDtypeStruct((B, 1), jnp.float32),
    )(num, emb2, *weights)


def kernel(numerical_features, categorical_features, embedding_tables,
           Wb0, bb0, Wb1, bb1, Wb2, bb2,
           Wt0, bt0, Wt1, bt1, Wt2, bt2, Wt3, bt3, Wt4, bt4):
    rowbase = (jnp.arange(NROWS, dtype=jnp.int32) * VOCABP)[:, None]
    idx_all = (rowbase
               + jnp.repeat(categorical_features.T, DIM, axis=0)).reshape(-1)
    tflat = _tc_repack(jnp.swapaxes(embedding_tables, 1, 2)).reshape(-1)
    emb_dm = _sc_gather(tflat, idx_all)
    bf = jnp.bfloat16
    emb2 = emb_dm.reshape(NROWS, B).T.astype(bf)

    w0x = Wt0[:DIM]
    w0f = jnp.zeros((NI * NI, Wt0.shape[1]), jnp.float32).at[_POS].set(Wt0[DIM:])
    r1 = lambda v: v.reshape(1, -1)
    return _tc_forward(
        numerical_features, emb2,
        Wb0.astype(bf), r1(bb0), Wb1.astype(bf), r1(bb1), Wb2.astype(bf),
        r1(bb2), w0x.astype(bf), w0f.astype(bf), r1(bt0), Wt1.astype(bf),
        r1(bt1), Wt2.astype(bf), r1(bt2), Wt3.astype(bf), r1(bt3),
        Wt4.astype(bf), r1(bt4))
